# baseline (speedup 1.0000x reference)
import functools

import jax
import jax.numpy as jnp
from jax import lax
from jax.experimental import pallas as pl
from jax.experimental.pallas import tpu as pltpu
from jax.experimental.pallas import tpu_sc as plsc

N = 50000
F = 128
E = 800000
NNZ = 250000
C = 40

NS = 16
W = 16

SL = 128
HC = 1536
HSTR = HC // SL
TRIPS = 34
NCH_E = NS * TRIPS
EP = NCH_E * HC
NCH_L = NCH_E + 2 * NS
EPL = NCH_L * HC

CHUNK = 1024
NSTR = CHUNK // SL
NNZP = ((NNZ + CHUNK - 1) // CHUNK) * CHUNK
NCH_F = NNZP // CHUNK

CP = 1000
NCP = N // CP

_mesh = plsc.VectorSubcoreMesh(core_axis_name="c", subcore_axis_name="s")
_sc_params = pltpu.CompilerParams(use_tc_tiling_on_sc=False)


def _make_hop():
  @functools.partial(
      pl.kernel,
      mesh=_mesh,
      compiler_params=_sc_params,
      out_type=jax.ShapeDtypeStruct((12 * N, W), jnp.float32),
      scratch_types=[
          pltpu.VMEM((HC,), jnp.float32),
          pltpu.VMEM((HC,), jnp.float32),
          pltpu.VMEM((HSTR, SL), jnp.int32),
          pltpu.VMEM((HSTR, SL), jnp.int32),
          pltpu.VMEM((HSTR, SL), jnp.int32),
          pltpu.VMEM((HSTR, SL), jnp.int32),
          pltpu.VMEM((HC, W), jnp.float32),
          pltpu.VMEM((HC, W), jnp.float32),
          pltpu.VMEM((CP, W), jnp.float32),
          pltpu.VMEM_SHARED((N, W), jnp.float32),
          pltpu.SemaphoreType.DMA,
          pltpu.SemaphoreType.DMA,
          pltpu.SemaphoreType.DMA,
          pltpu.SemaphoreType.DMA,
      ],
  )
  def hop(x_hbm, rows3_hbm, cols3_hbm, vals_hbm, out_hbm,
          val_a, val_b, gidx_a, gidx_b, sidx_a, sidx_b, gbuf_a, gbuf_b,
          zbuf, acc, lsem, xsem, gsem, ssem):
    c = lax.axis_index("c")
    s = lax.axis_index("s")
    bufs = ((val_a, gidx_a, sidx_a, gbuf_a), (val_b, gidx_b, sidx_b, gbuf_b))

    def zfill(i, _):
      zbuf[i, pl.ds(0, W)] = jnp.zeros((W,), jnp.float32)
      return _
    lax.fori_loop(0, CP, zfill, None)

    ctrips = jnp.where(s < NCP % NS, NCP // NS + 1, NCP // NS)

    for r in range(6):
      if r < 4:
        out_chunk = c + 2 * r
        chunk_in = c + 2 * r
        table = x_hbm
      else:
        out_chunk = 8 + c + 2 * (r - 4)
        chunk_in = 4 + c + 2 * (r - 4)
        table = out_hbm

      def fire_gv(k, p):
        v, gi, _, _ = bufs[p]
        off = (s + NS * k) * HC
        bj = (s + NS * k) * HSTR
        pltpu.async_copy(cols3_hbm.at[chunk_in, pl.ds(bj, HSTR)], gi, lsem)
        pltpu.async_copy(vals_hbm.at[pl.ds(off, HC)], v, lsem)

      def wait_gv(p):
        v, gi, _, _ = bufs[p]
        pltpu.make_async_copy(rows3_hbm.at[pl.ds(0, HSTR)], gi, lsem).wait()
        pltpu.make_async_copy(vals_hbm.at[pl.ds(0, HC)], v, lsem).wait()

      def fire_sidx(k, p):
        si = bufs[p][2]
        bj = (s + NS * k) * HSTR
        pltpu.async_copy(rows3_hbm.at[pl.ds(bj, HSTR)], si, xsem)

      def wait_sidx(p):
        si = bufs[p][2]
        pltpu.make_async_copy(rows3_hbm.at[pl.ds(0, HSTR)], si, xsem).wait()

      def fire_gathers(p):
        _, gi, _, gb = bufs[p]
        for j in range(HSTR):
          pltpu.async_copy(table.at[gi.at[j]], gb.at[pl.ds(j * SL, SL)], gsem)

      def wait_gathers(p):
        gb = bufs[p][3]
        pltpu.make_async_copy(x_hbm.at[pl.ds(0, HC)], gb, gsem).wait()

      def scale(p):
        v, _, _, gb = bufs[p]

        @plsc.parallel_loop(0, HC // 16, unroll=2)
        def body(g):
          v16 = v[pl.ds(g * 16, 16)]
          for i in range(16):
            e = g * 16 + i
            vv = jnp.full((W,), v16[i])
            gb[e, pl.ds(0, W)] = gb[e, pl.ds(0, W)] * vv

      def fire_scatters(p):
        _, _, si, gb = bufs[p]
        for j in range(HSTR):
          pltpu.async_copy(gb.at[pl.ds(j * SL, SL)], acc.at[si.at[j]], ssem,
                           add=True)

      def wait_scatters(p):
        gb = bufs[p][3]
        pltpu.make_async_copy(gb, acc.at[pl.ds(0, HC)], ssem).wait()

      def zacc(k, _):
        pltpu.sync_copy(zbuf, acc.at[pl.ds((s + NS * k) * CP, CP)])
        return _
      lax.fori_loop(0, ctrips, zacc, None)
      plsc.subcore_barrier()

      fire_gv(0, 0)
      fire_sidx(0, 0)
      wait_gv(0)
      fire_gathers(0)
      fire_gv(1, 1)
      fire_sidx(1, 1)
      wait_gathers(0)
      wait_gv(1)
      fire_gathers(1)
      scale(0)
      wait_sidx(0)
      fire_scatters(0)
      fire_gv(2, 0)
      wait_gathers(1)
      wait_scatters(0)
      wait_gv(0)
      fire_gathers(0)
      fire_sidx(2, 0)
      scale(1)
      wait_sidx(1)
      fire_scatters(1)
      fire_gv(3, 1)

      def pair(k2, _):
        for p in (0, 1):
          k = 2 + 2 * k2 + p
          wait_gathers(p)
          wait_scatters(1 - p)
          wait_gv(1 - p)
          fire_gathers(1 - p)
          fire_sidx(k + 1, 1 - p)
          scale(p)
          wait_sidx(p)
          fire_scatters(p)
          fire_gv(k + 2, p)
        return _
      lax.fori_loop(0, (TRIPS - 2) // 2, pair, None)

      wait_scatters(1)
      wait_gathers(0)
      wait_sidx(0)
      wait_gv(0)
      plsc.subcore_barrier()

      def cpo(k, _):
        base = (s + NS * k) * CP
        pltpu.sync_copy(acc.at[pl.ds(base, CP)], gbuf_a.at[pl.ds(0, CP)])
        pltpu.sync_copy(gbuf_a.at[pl.ds(0, CP)],
                        out_hbm.at[pl.ds(out_chunk * N + base, CP)])
        return _
      lax.fori_loop(0, ctrips, cpo, None)

  return hop


_hop6 = _make_hop()


@functools.partial(
    pl.kernel,
    mesh=_mesh,
    compiler_params=_sc_params,
    out_type=jax.ShapeDtypeStruct((8 * N * W,), jnp.float32),
    scratch_types=[
        pltpu.VMEM((CHUNK,), jnp.int32),
        pltpu.VMEM((CHUNK,), jnp.int32),
        pltpu.VMEM((CHUNK,), jnp.float32),
        pltpu.VMEM((NSTR, SL), jnp.int32),
        pltpu.VMEM((NSTR, SL), jnp.float32),
        pltpu.VMEM((10000,), jnp.float32),
        pltpu.VMEM((10000,), jnp.float32),
        pltpu.VMEM_SHARED((N * W,), jnp.float32),
        pltpu.SemaphoreType.DMA,
        pltpu.SemaphoreType.DMA,
    ],
)
def _densify(rows_hbm, cols_hbm, vals_hbm, out_hbm,
             row_v, col_v, val_v, sidx, vbuf, zbuf, bbuf, acc, lsem, ssem):
  c = lax.axis_index("c")
  s = lax.axis_index("s")
  wps = (N * W) // NS

  def zfill(i, _):
    zbuf[pl.ds(i * 16, 16)] = jnp.zeros((16,), jnp.float32)
    return _
  lax.fori_loop(0, 625, zfill, None)

  ntrips = jnp.where(s < NCH_F % NS, NCH_F // NS + 1, NCH_F // NS)

  for r in range(4):
    chunk = c + 2 * r
    colbase = W * chunk

    def zacc(k, _):
      pltpu.sync_copy(zbuf, acc.at[pl.ds(s * wps + k * 10000, 10000)])
      return _
    lax.fori_loop(0, wps // 10000, zacc, None)
    plsc.subcore_barrier()

    def batch(k, _):
      off = (s + NS * k) * CHUNK
      pltpu.async_copy(rows_hbm.at[pl.ds(off, CHUNK)], row_v, lsem)
      pltpu.async_copy(cols_hbm.at[pl.ds(off, CHUNK)], col_v, lsem)
      pltpu.async_copy(vals_hbm.at[pl.ds(off, CHUNK)], val_v, lsem)
      pltpu.make_async_copy(rows_hbm.at[pl.ds(0, CHUNK)], row_v, lsem).wait()
      pltpu.make_async_copy(rows_hbm.at[pl.ds(0, CHUNK)], col_v, lsem).wait()
      pltpu.make_async_copy(vals_hbm.at[pl.ds(0, CHUNK)], val_v, lsem).wait()

      def bidx(j, _):
        for l in range(SL // 16):
          src = pl.ds(j * SL + l * 16, 16)
          r16 = row_v[src]
          c16 = col_v[src]
          v16 = val_v[src]
          inr = (c16 >= colbase) & (c16 < colbase + W)
          flat = r16 * W + (c16 - colbase)
          alt = r16 * W + (c16 & (W - 1))
          sidx[j, pl.ds(l * 16, 16)] = jnp.where(inr, flat, alt)
          vbuf[j, pl.ds(l * 16, 16)] = jnp.where(inr, v16, 0.0)
        return _
      lax.fori_loop(0, NSTR, bidx, None)

      for j in range(NSTR):
        pltpu.async_copy(vbuf.at[j], acc.at[sidx.at[j]], ssem, add=True)
      for j in range(NSTR):
        pltpu.make_async_copy(vbuf.at[j], acc.at[pl.ds(0, SL)], ssem).wait()
      return _
    lax.fori_loop(0, ntrips, batch, None)
    plsc.subcore_barrier()

    def cpo(k, _):
      pltpu.sync_copy(acc.at[pl.ds(s * wps + k * 10000, 10000)], bbuf)
      pltpu.sync_copy(
          bbuf,
          out_hbm.at[pl.ds(chunk * (N * W) + s * wps + k * 10000, 10000)])
      return _
    lax.fori_loop(0, wps // 10000, cpo, None)


TB = 2000
NB = N // TB


def _tc_up(x8, wcat, bcat):
  def body(x_ref, w_ref, b_ref, s0_ref, sp_ref):
    x = jnp.concatenate([x_ref[k] for k in range(8)], axis=1)
    h = jnp.dot(x, w_ref[...], preferred_element_type=jnp.float32)
    h = jnp.maximum(h + b_ref[...], 0.0)
    s0_ref[...] = h[:, :64]
    for k in range(8):
      sp_ref[k] = h[:, 64 + W * k:64 + W * (k + 1)]

  return pl.pallas_call(
      body,
      grid=(NB,),
      in_specs=[
          pl.BlockSpec((8, TB, W), lambda i: (0, i, 0)),
          pl.BlockSpec((F, 192), lambda i: (0, 0)),
          pl.BlockSpec((1, 192), lambda i: (0, 0)),
      ],
      out_specs=[
          pl.BlockSpec((TB, 64), lambda i: (i, 0)),
          pl.BlockSpec((8, TB, W), lambda i: (0, i, 0)),
      ],
      out_shape=[
          jax.ShapeDtypeStruct((N, 64), jnp.float32),
          jax.ShapeDtypeStruct((8, N, W), jnp.float32),
      ],
  )(x8, wcat, bcat)


def _tc_mid(s0, h12, h3, wcat):
  def body(s0_ref, h12_ref, h3_ref, w_ref, t0_ref, tp_ref):
    af1 = jnp.concatenate(
        [s0_ref[...]] + [h12_ref[k] for k in range(4)]
        + [h3_ref[k] for k in range(4)], axis=1)
    t = jnp.dot(af1, w_ref[...], preferred_element_type=jnp.float32)
    t0_ref[...] = t[:, :64]
    for k in range(8):
      tp_ref[k] = t[:, 64 + W * k:64 + W * (k + 1)]

  return pl.pallas_call(
      body,
      grid=(NB,),
      in_specs=[
          pl.BlockSpec((TB, 64), lambda i: (i, 0)),
          pl.BlockSpec((4, TB, W), lambda i: (0, i, 0)),
          pl.BlockSpec((4, TB, W), lambda i: (2, i, 0)),
          pl.BlockSpec((192, 192), lambda i: (0, 0)),
      ],
      out_specs=[
          pl.BlockSpec((TB, 64), lambda i: (i, 0)),
          pl.BlockSpec((8, TB, W), lambda i: (0, i, 0)),
      ],
      out_shape=[
          jax.ShapeDtypeStruct((N, 64), jnp.float32),
          jax.ShapeDtypeStruct((8, N, W), jnp.float32),
      ],
  )(s0, h12, h3, wcat)


def _tc_fin(t0, u12, u3, wfc, bbot, bfc):
  def body(t0_ref, u12_ref, u3_ref, w_ref, bb_ref, bf_ref, out_ref):
    af2 = jnp.concatenate(
        [t0_ref[...]] + [u12_ref[k] for k in range(4)]
        + [u3_ref[k] for k in range(4)], axis=1)
    w = w_ref[...]
    logits = (jnp.dot(af2, w, preferred_element_type=jnp.float32)
              + jnp.dot(bb_ref[...], w, preferred_element_type=jnp.float32)
              + bf_ref[...])
    m = jnp.max(logits, axis=1, keepdims=True)
    ex = jnp.exp(logits - m)
    lse = m + jnp.log(jnp.sum(ex, axis=1, keepdims=True))
    out_ref[...] = logits - lse

  return pl.pallas_call(
      body,
      grid=(NB,),
      in_specs=[
          pl.BlockSpec((TB, 64), lambda i: (i, 0)),
          pl.BlockSpec((4, TB, W), lambda i: (0, i, 0)),
          pl.BlockSpec((4, TB, W), lambda i: (2, i, 0)),
          pl.BlockSpec((192, C), lambda i: (0, 0)),
          pl.BlockSpec((1, 192), lambda i: (0, 0)),
          pl.BlockSpec((1, C), lambda i: (0, 0)),
      ],
      out_specs=pl.BlockSpec((TB, C), lambda i: (i, 0)),
      out_shape=jax.ShapeDtypeStruct((N, C), jnp.float32),
  )(t0, u12, u3, wfc, bbot, bfc)


def kernel(features_indices, features_values, adj_indices, adj_values,
           W_up0, b_up0, W_up1, b_up1, W_up2, b_up2,
           W_bot0, b_bot0, W_bot1, b_bot1, W_bot2, b_bot2,
           W_fc, b_fc):
  i32 = jnp.int32
  frows = features_indices[0].astype(i32)
  fcols = features_indices[1].astype(i32)
  arows = adj_indices[0].astype(i32)
  acols = adj_indices[1].astype(i32)

  epad = EPL - E
  spread_e = jnp.arange(epad, dtype=i32) % N
  arows_p = jnp.concatenate([arows, spread_e])
  acols_p = jnp.concatenate([acols, spread_e])
  avals_p = jnp.concatenate([adj_values, jnp.zeros((epad,), jnp.float32)])
  rows3 = arows_p.reshape(NCH_L * HSTR, SL)
  cols3 = (acols_p[None, :]
           + (jnp.arange(8, dtype=i32) * N)[:, None]).reshape(
               8, NCH_L * HSTR, SL)

  fpad = NNZP - NNZ
  spread_f = jnp.arange(fpad, dtype=i32) % N
  frows_p = jnp.concatenate([frows, spread_f])
  fcols_p = jnp.concatenate([fcols, jnp.zeros((fpad,), i32)])
  fvals_p = jnp.concatenate([features_values, jnp.zeros((fpad,), jnp.float32)])

  wupc = jnp.concatenate([W_up0, W_up1, W_up2], axis=1)
  bupc = jnp.concatenate([b_up0, b_up1, b_up2], axis=1)
  wbotc = jnp.concatenate([W_bot0, W_bot1, W_bot2], axis=1)
  bbotc = jnp.concatenate([b_bot0, b_bot1, b_bot2], axis=1)
  bfc2 = b_fc.reshape(1, C)

  x8flat = _densify(frows_p, fcols_p, fvals_p)
  x8 = x8flat.reshape(8, N, W)

  s0, sp = _tc_up(x8, wupc, bupc)
  sp_flat = sp.reshape(8 * N, W)

  h_flat = _hop6(sp_flat, rows3, cols3, avals_p)
  h = h_flat.reshape(12, N, W)

  t0, tp = _tc_mid(s0, h, h, wbotc)
  tp_flat = tp.reshape(8 * N, W)

  u_flat = _hop6(tp_flat, rows3, cols3, avals_p)
  u = u_flat.reshape(12, N, W)

  return _tc_fin(t0, u, u, W_fc, bbotc, bfc2)


# speedup vs baseline: 1.0678x; 1.0678x over previous
import functools

import jax
import jax.numpy as jnp
from jax import lax
from jax.experimental import pallas as pl
from jax.experimental.pallas import tpu as pltpu
from jax.experimental.pallas import tpu_sc as plsc

N = 50000
F = 128
E = 800000
NNZ = 250000
C = 40

NS = 16
W = 16

SL = 128
HC = 1536
HSTR = HC // SL
TRIPS = 34
NCH_E = NS * TRIPS
EP = NCH_E * HC
NCH_L = NCH_E + 2 * NS
EPL = NCH_L * HC

CHUNK = 1024
NSTR = CHUNK // SL
NNZP = ((NNZ + CHUNK - 1) // CHUNK) * CHUNK
NCH_F = NNZP // CHUNK

CP = 1000
NCP = N // CP

_mesh = plsc.VectorSubcoreMesh(core_axis_name="c", subcore_axis_name="s")
_sc_params = pltpu.CompilerParams(use_tc_tiling_on_sc=False)


def _make_hop(num_rounds, in_off, out_chunks):
  @functools.partial(
      pl.kernel,
      mesh=_mesh,
      compiler_params=_sc_params,
      out_type=jax.ShapeDtypeStruct((out_chunks * N, W), jnp.float32),
      scratch_types=[
          pltpu.VMEM((HC,), jnp.float32),
          pltpu.VMEM((HC,), jnp.float32),
          pltpu.VMEM((HSTR, SL), jnp.int32),
          pltpu.VMEM((HSTR, SL), jnp.int32),
          pltpu.VMEM((HSTR, SL), jnp.int32),
          pltpu.VMEM((HSTR, SL), jnp.int32),
          pltpu.VMEM((HC, W), jnp.float32),
          pltpu.VMEM((HC, W), jnp.float32),
          pltpu.VMEM((CP, W), jnp.float32),
          pltpu.VMEM_SHARED((N, W), jnp.float32),
          pltpu.SemaphoreType.DMA,
          pltpu.SemaphoreType.DMA,
          pltpu.SemaphoreType.DMA,
          pltpu.SemaphoreType.DMA,
      ],
  )
  def hop(x_hbm, rows3_hbm, cols3_hbm, vals_hbm, out_hbm,
          val_a, val_b, gidx_a, gidx_b, sidx_a, sidx_b, gbuf_a, gbuf_b,
          zbuf, acc, lsem, xsem, gsem, ssem):
    c = lax.axis_index("c")
    s = lax.axis_index("s")
    bufs = ((val_a, gidx_a, sidx_a, gbuf_a), (val_b, gidx_b, sidx_b, gbuf_b))

    def zfill(i, _):
      zbuf[i, pl.ds(0, W)] = jnp.zeros((W,), jnp.float32)
      return _
    lax.fori_loop(0, CP, zfill, None)

    ctrips = jnp.where(s < NCP % NS, NCP // NS + 1, NCP // NS)

    for r in range(num_rounds):
      out_chunk = c + 2 * r
      chunk_in = in_off + c + 2 * r
      table = x_hbm

      def fire_gv(k, p):
        v, gi, _, _ = bufs[p]
        off = (s + NS * k) * HC
        bj = (s + NS * k) * HSTR
        pltpu.async_copy(cols3_hbm.at[chunk_in, pl.ds(bj, HSTR)], gi, lsem)
        pltpu.async_copy(vals_hbm.at[pl.ds(off, HC)], v, lsem)

      def wait_gv(p):
        v, gi, _, _ = bufs[p]
        pltpu.make_async_copy(rows3_hbm.at[pl.ds(0, HSTR)], gi, lsem).wait()
        pltpu.make_async_copy(vals_hbm.at[pl.ds(0, HC)], v, lsem).wait()

      def fire_sidx(k, p):
        si = bufs[p][2]
        bj = (s + NS * k) * HSTR
        pltpu.async_copy(rows3_hbm.at[pl.ds(bj, HSTR)], si, xsem)

      def wait_sidx(p):
        si = bufs[p][2]
        pltpu.make_async_copy(rows3_hbm.at[pl.ds(0, HSTR)], si, xsem).wait()

      def fire_gathers(p):
        _, gi, _, gb = bufs[p]
        for j in range(HSTR):
          pltpu.async_copy(table.at[gi.at[j]], gb.at[pl.ds(j * SL, SL)], gsem)

      def wait_gathers(p):
        gb = bufs[p][3]
        pltpu.make_async_copy(x_hbm.at[pl.ds(0, HC)], gb, gsem).wait()

      def scale(p):
        v, _, _, gb = bufs[p]

        @plsc.parallel_loop(0, HC // 16, unroll=2)
        def body(g):
          v16 = v[pl.ds(g * 16, 16)]
          for i in range(16):
            e = g * 16 + i
            vv = jnp.full((W,), v16[i])
            gb[e, pl.ds(0, W)] = gb[e, pl.ds(0, W)] * vv

      def fire_scatters(p):
        _, _, si, gb = bufs[p]
        for j in range(HSTR):
          pltpu.async_copy(gb.at[pl.ds(j * SL, SL)], acc.at[si.at[j]], ssem,
                           add=True)

      def wait_scatters(p):
        gb = bufs[p][3]
        pltpu.make_async_copy(gb, acc.at[pl.ds(0, HC)], ssem).wait()

      def zacc(k, _):
        pltpu.sync_copy(zbuf, acc.at[pl.ds((s + NS * k) * CP, CP)])
        return _
      lax.fori_loop(0, ctrips, zacc, None)
      plsc.subcore_barrier()

      fire_gv(0, 0)
      fire_sidx(0, 0)
      wait_gv(0)
      fire_gathers(0)
      fire_gv(1, 1)
      fire_sidx(1, 1)
      wait_gathers(0)
      wait_gv(1)
      fire_gathers(1)
      scale(0)
      wait_sidx(0)
      fire_scatters(0)
      fire_gv(2, 0)
      wait_gathers(1)
      wait_scatters(0)
      wait_gv(0)
      fire_gathers(0)
      fire_sidx(2, 0)
      scale(1)
      wait_sidx(1)
      fire_scatters(1)
      fire_gv(3, 1)

      def pair(k2, _):
        for p in (0, 1):
          k = 2 + 2 * k2 + p
          wait_gathers(p)
          wait_scatters(1 - p)
          wait_gv(1 - p)
          fire_gathers(1 - p)
          fire_sidx(k + 1, 1 - p)
          scale(p)
          wait_sidx(p)
          fire_scatters(p)
          fire_gv(k + 2, p)
        return _
      lax.fori_loop(0, (TRIPS - 2) // 2, pair, None)

      wait_scatters(1)
      wait_gathers(0)
      wait_sidx(0)
      wait_gv(0)
      plsc.subcore_barrier()

      def cpo(k, _):
        base = (s + NS * k) * CP
        pltpu.sync_copy(acc.at[pl.ds(base, CP)], gbuf_a.at[pl.ds(0, CP)])
        pltpu.sync_copy(gbuf_a.at[pl.ds(0, CP)],
                        out_hbm.at[pl.ds(out_chunk * N + base, CP)])
        return _
      lax.fori_loop(0, ctrips, cpo, None)

  return hop


_hop4 = _make_hop(4, 0, 8)
_hop2 = _make_hop(2, 4, 4)


@functools.partial(
    pl.kernel,
    mesh=_mesh,
    compiler_params=_sc_params,
    out_type=jax.ShapeDtypeStruct((8 * N * W,), jnp.float32),
    scratch_types=[
        pltpu.VMEM((CHUNK,), jnp.int32),
        pltpu.VMEM((CHUNK,), jnp.int32),
        pltpu.VMEM((CHUNK,), jnp.float32),
        pltpu.VMEM((NSTR, SL), jnp.int32),
        pltpu.VMEM((NSTR, SL), jnp.float32),
        pltpu.VMEM((10000,), jnp.float32),
        pltpu.VMEM((10000,), jnp.float32),
        pltpu.VMEM_SHARED((N * W,), jnp.float32),
        pltpu.SemaphoreType.DMA,
        pltpu.SemaphoreType.DMA,
    ],
)
def _densify(rows_hbm, cols_hbm, vals_hbm, out_hbm,
             row_v, col_v, val_v, sidx, vbuf, zbuf, bbuf, acc, lsem, ssem):
  c = lax.axis_index("c")
  s = lax.axis_index("s")
  wps = (N * W) // NS

  def zfill(i, _):
    zbuf[pl.ds(i * 16, 16)] = jnp.zeros((16,), jnp.float32)
    return _
  lax.fori_loop(0, 625, zfill, None)

  ntrips = jnp.where(s < NCH_F % NS, NCH_F // NS + 1, NCH_F // NS)

  for r in range(4):
    chunk = c + 2 * r
    colbase = W * chunk

    def zacc(k, _):
      pltpu.sync_copy(zbuf, acc.at[pl.ds(s * wps + k * 10000, 10000)])
      return _
    lax.fori_loop(0, wps // 10000, zacc, None)
    plsc.subcore_barrier()

    def batch(k, _):
      off = (s + NS * k) * CHUNK
      pltpu.async_copy(rows_hbm.at[pl.ds(off, CHUNK)], row_v, lsem)
      pltpu.async_copy(cols_hbm.at[pl.ds(off, CHUNK)], col_v, lsem)
      pltpu.async_copy(vals_hbm.at[pl.ds(off, CHUNK)], val_v, lsem)
      pltpu.make_async_copy(rows_hbm.at[pl.ds(0, CHUNK)], row_v, lsem).wait()
      pltpu.make_async_copy(rows_hbm.at[pl.ds(0, CHUNK)], col_v, lsem).wait()
      pltpu.make_async_copy(vals_hbm.at[pl.ds(0, CHUNK)], val_v, lsem).wait()

      def bidx(j, _):
        for l in range(SL // 16):
          src = pl.ds(j * SL + l * 16, 16)
          r16 = row_v[src]
          c16 = col_v[src]
          v16 = val_v[src]
          inr = (c16 >= colbase) & (c16 < colbase + W)
          flat = r16 * W + (c16 - colbase)
          alt = r16 * W + (c16 & (W - 1))
          sidx[j, pl.ds(l * 16, 16)] = jnp.where(inr, flat, alt)
          vbuf[j, pl.ds(l * 16, 16)] = jnp.where(inr, v16, 0.0)
        return _
      lax.fori_loop(0, NSTR, bidx, None)

      for j in range(NSTR):
        pltpu.async_copy(vbuf.at[j], acc.at[sidx.at[j]], ssem, add=True)
      for j in range(NSTR):
        pltpu.make_async_copy(vbuf.at[j], acc.at[pl.ds(0, SL)], ssem).wait()
      return _
    lax.fori_loop(0, ntrips, batch, None)
    plsc.subcore_barrier()

    def cpo(k, _):
      pltpu.sync_copy(acc.at[pl.ds(s * wps + k * 10000, 10000)], bbuf)
      pltpu.sync_copy(
          bbuf,
          out_hbm.at[pl.ds(chunk * (N * W) + s * wps + k * 10000, 10000)])
      return _
    lax.fori_loop(0, wps // 10000, cpo, None)


TB = 2000
NB = N // TB


def _tc_up(x8, wcat, bcat):
  def body(x_ref, w_ref, b_ref, s0_ref, sp_ref):
    x = jnp.concatenate([x_ref[k] for k in range(8)], axis=1)
    h = jnp.dot(x, w_ref[...], preferred_element_type=jnp.float32)
    h = jnp.maximum(h + b_ref[...], 0.0)
    s0_ref[...] = h[:, :64]
    for k in range(8):
      sp_ref[k] = h[:, 64 + W * k:64 + W * (k + 1)]

  return pl.pallas_call(
      body,
      grid=(NB,),
      in_specs=[
          pl.BlockSpec((8, TB, W), lambda i: (0, i, 0)),
          pl.BlockSpec((F, 192), lambda i: (0, 0)),
          pl.BlockSpec((1, 192), lambda i: (0, 0)),
      ],
      out_specs=[
          pl.BlockSpec((TB, 64), lambda i: (i, 0)),
          pl.BlockSpec((8, TB, W), lambda i: (0, i, 0)),
      ],
      out_shape=[
          jax.ShapeDtypeStruct((N, 64), jnp.float32),
          jax.ShapeDtypeStruct((8, N, W), jnp.float32),
      ],
  )(x8, wcat, bcat)


def _tc_mid(s0, h12, h3, wcat):
  def body(s0_ref, h12_ref, h3_ref, w_ref, t0_ref, tp_ref):
    af1 = jnp.concatenate(
        [s0_ref[...]] + [h12_ref[k] for k in range(4)]
        + [h3_ref[k] for k in range(4)], axis=1)
    t = jnp.dot(af1, w_ref[...], preferred_element_type=jnp.float32)
    t0_ref[...] = t[:, :64]
    for k in range(8):
      tp_ref[k] = t[:, 64 + W * k:64 + W * (k + 1)]

  return pl.pallas_call(
      body,
      grid=(NB,),
      in_specs=[
          pl.BlockSpec((TB, 64), lambda i: (i, 0)),
          pl.BlockSpec((4, TB, W), lambda i: (0, i, 0)),
          pl.BlockSpec((4, TB, W), lambda i: (0, i, 0)),
          pl.BlockSpec((192, 192), lambda i: (0, 0)),
      ],
      out_specs=[
          pl.BlockSpec((TB, 64), lambda i: (i, 0)),
          pl.BlockSpec((8, TB, W), lambda i: (0, i, 0)),
      ],
      out_shape=[
          jax.ShapeDtypeStruct((N, 64), jnp.float32),
          jax.ShapeDtypeStruct((8, N, W), jnp.float32),
      ],
  )(s0, h12, h3, wcat)


def _tc_fin(t0, u12, u3, wfc, bbot, bfc):
  def body(t0_ref, u12_ref, u3_ref, w_ref, bb_ref, bf_ref, out_ref):
    af2 = jnp.concatenate(
        [t0_ref[...]] + [u12_ref[k] for k in range(4)]
        + [u3_ref[k] for k in range(4)], axis=1)
    w = w_ref[...]
    logits = (jnp.dot(af2, w, preferred_element_type=jnp.float32)
              + jnp.dot(bb_ref[...], w, preferred_element_type=jnp.float32)
              + bf_ref[...])
    m = jnp.max(logits, axis=1, keepdims=True)
    ex = jnp.exp(logits - m)
    lse = m + jnp.log(jnp.sum(ex, axis=1, keepdims=True))
    out_ref[...] = logits - lse

  return pl.pallas_call(
      body,
      grid=(NB,),
      in_specs=[
          pl.BlockSpec((TB, 64), lambda i: (i, 0)),
          pl.BlockSpec((4, TB, W), lambda i: (0, i, 0)),
          pl.BlockSpec((4, TB, W), lambda i: (0, i, 0)),
          pl.BlockSpec((192, C), lambda i: (0, 0)),
          pl.BlockSpec((1, 192), lambda i: (0, 0)),
          pl.BlockSpec((1, C), lambda i: (0, 0)),
      ],
      out_specs=pl.BlockSpec((TB, C), lambda i: (i, 0)),
      out_shape=jax.ShapeDtypeStruct((N, C), jnp.float32),
  )(t0, u12, u3, wfc, bbot, bfc)


def kernel(features_indices, features_values, adj_indices, adj_values,
           W_up0, b_up0, W_up1, b_up1, W_up2, b_up2,
           W_bot0, b_bot0, W_bot1, b_bot1, W_bot2, b_bot2,
           W_fc, b_fc):
  i32 = jnp.int32
  frows = features_indices[0].astype(i32)
  fcols = features_indices[1].astype(i32)
  arows = adj_indices[0].astype(i32)
  acols = adj_indices[1].astype(i32)

  epad = EPL - E
  spread_e = jnp.arange(epad, dtype=i32) % N
  arows_p = jnp.concatenate([arows, spread_e])
  acols_p = jnp.concatenate([acols, spread_e])
  avals_p = jnp.concatenate([adj_values, jnp.zeros((epad,), jnp.float32)])
  rows3 = arows_p.reshape(NCH_L * HSTR, SL)
  cols3 = (acols_p[None, :]
           + (jnp.arange(8, dtype=i32) * N)[:, None]).reshape(
               8, NCH_L * HSTR, SL)

  fpad = NNZP - NNZ
  spread_f = jnp.arange(fpad, dtype=i32) % N
  frows_p = jnp.concatenate([frows, spread_f])
  fcols_p = jnp.concatenate([fcols, jnp.zeros((fpad,), i32)])
  fvals_p = jnp.concatenate([features_values, jnp.zeros((fpad,), jnp.float32)])

  wupc = jnp.concatenate([W_up0, W_up1, W_up2], axis=1)
  bupc = jnp.concatenate([b_up0, b_up1, b_up2], axis=1)
  wbotc = jnp.concatenate([W_bot0, W_bot1, W_bot2], axis=1)
  bbotc = jnp.concatenate([b_bot0, b_bot1, b_bot2], axis=1)
  bfc2 = b_fc.reshape(1, C)

  x8flat = _densify(frows_p, fcols_p, fvals_p)
  x8 = x8flat.reshape(8, N, W)

  s0, sp = _tc_up(x8, wupc, bupc)
  sp_flat = sp.reshape(8 * N, W)

  h12_flat = _hop4(sp_flat, rows3, cols3, avals_p)
  h3_flat = _hop2(h12_flat, rows3, cols3, avals_p)
  h12 = h12_flat.reshape(8, N, W)
  h3 = h3_flat.reshape(4, N, W)

  t0, tp = _tc_mid(s0, h12, h3, wbotc)
  tp_flat = tp.reshape(8 * N, W)

  u12_flat = _hop4(tp_flat, rows3, cols3, avals_p)
  u3_flat = _hop2(u12_flat, rows3, cols3, avals_p)
  u12 = u12_flat.reshape(8, N, W)
  u3 = u3_flat.reshape(4, N, W)

  return _tc_fin(t0, u12, u3, W_fc, bbotc, bfc2)
